# baseline (speedup 1.0000x reference)
import functools

import jax
import jax.numpy as jnp
from jax import lax
from jax.experimental import pallas as pl
from jax.experimental.pallas import tpu as pltpu
from jax.experimental.pallas import tpu_sc as plsc

_N_SLICES = 4


def _transpose_pad_body(dim, t_ref, out_ref):
    out_ref[:, :dim] = t_ref[...].T
    out_ref[:, dim:] = jnp.zeros_like(out_ref[:, dim:])


def _transpose_pad(table_t, block_rows=2048):
    dim, vocab = table_t.shape
    grid = pl.cdiv(vocab, block_rows)
    return pl.pallas_call(
        functools.partial(_transpose_pad_body, dim),
        grid=(grid,),
        in_specs=[pl.BlockSpec((dim, block_rows), lambda i: (0, i))],
        out_specs=pl.BlockSpec((block_rows, 384), lambda i: (i, 0)),
        out_shape=jax.ShapeDtypeStruct((vocab, 384), jnp.float32),
    )(table_t)


def _make_gather(slice_idx, n_workers, n_chunks, chunk, num_cores):
    mesh = plsc.VectorSubcoreMesh(core_axis_name="c", subcore_axis_name="s")

    @functools.partial(
        pl.kernel,
        mesh=mesh,
        out_type=jax.ShapeDtypeStruct((n_workers * n_chunks * chunk, 384),
                                      jnp.float32),
        scratch_types=[
            pltpu.VMEM((n_chunks, chunk), jnp.int32),
            pltpu.VMEM((2, chunk, 384), jnp.float32),
            pltpu.SemaphoreType.DMA,
            pltpu.SemaphoreType.DMA,
            pltpu.SemaphoreType.DMA,
            pltpu.SemaphoreType.DMA,
        ],
    )
    def gather_kernel(idx_hbm, table_hbm, prev_hbm, out_hbm, idx_v, row_v,
                      gsem0, gsem1, wsem0, wsem1):
        del prev_hbm
        wid = lax.axis_index("s") * num_cores + lax.axis_index("c")
        pltpu.sync_copy(idx_hbm.at[slice_idx * n_workers + wid], idx_v)
        base = wid * (n_chunks * chunk)
        gsems = (gsem0, gsem1)
        wsems = (wsem0, wsem1)
        gathers = [None, None]
        writes = [None, None]
        gathers[0] = pltpu.async_copy(
            table_hbm.at[idx_v.at[0]], row_v.at[0], gsems[0])
        for j in range(n_chunks):
            cur = j % 2
            nxt = 1 - cur
            if j + 1 < n_chunks:
                if writes[nxt] is not None:
                    writes[nxt].wait()
                gathers[nxt] = pltpu.async_copy(
                    table_hbm.at[idx_v.at[j + 1]], row_v.at[nxt], gsems[nxt])
            gathers[cur].wait()
            writes[cur] = pltpu.async_copy(
                row_v.at[cur], out_hbm.at[pl.ds(base + j * chunk, chunk)],
                wsems[cur])
        for w in writes:
            if w is not None:
                w.wait()

    return gather_kernel


def _transpose_merge_body(dim, wide_ref, _acc_ref, out_ref):
    out_ref[0, ...] = wide_ref[:, :dim].T


def _merge_slice(wide_k, acc, k, seq_s, seq, batch, dim, block_cols=2048):
    nb = batch // block_cols
    grid = (seq_s, nb)
    return pl.pallas_call(
        functools.partial(_transpose_merge_body, dim),
        grid=grid,
        in_specs=[
            pl.BlockSpec((block_cols, 384), lambda s, b: (s * nb + b, 0)),
            pl.BlockSpec(memory_space=pl.ANY),
        ],
        out_specs=pl.BlockSpec((1, dim, block_cols),
                               lambda s, b, k=k: (k * seq_s + s, 0, b)),
        out_shape=jax.ShapeDtypeStruct((seq, dim, batch), jnp.float32),
        input_output_aliases={1: 0},
    )(wide_k, acc)


def _merge_first_body(dim, k, seq_s, nb, wide_ref, out_ref):
    out_ref[0, ...] = wide_ref[:, :dim].T


def _merge_first(wide_k, seq_s, seq, batch, dim, block_cols=2048):
    nb = batch // block_cols
    grid = (seq_s, nb)
    return pl.pallas_call(
        functools.partial(_merge_first_body, dim, 0, seq_s, nb),
        grid=grid,
        in_specs=[pl.BlockSpec((block_cols, 384), lambda s, b: (s * nb + b, 0))],
        out_specs=pl.BlockSpec((1, dim, block_cols), lambda s, b: (s, 0, b)),
        out_shape=jax.ShapeDtypeStruct((seq, dim, batch), jnp.float32),
    )(wide_k)


def kernel(inputs, table):
    batch, seq = inputs.shape
    vocab, dim = table.shape
    total = batch * seq

    info = plsc.get_sparse_core_info()
    n_workers = info.num_cores * info.num_subcores
    n_slices = _N_SLICES
    per_worker = total // (n_slices * n_workers)
    chunk = 128
    n_chunks = per_worker // chunk
    seq_s = seq // n_slices

    idx = inputs.astype(jnp.int32).T.reshape(
        n_slices * n_workers, n_chunks, chunk)
    table_wide = _transpose_pad(table.T)

    wides = []
    prev = idx
    for k in range(n_slices):
        fn = _make_gather(k, n_workers, n_chunks, chunk, info.num_cores)
        prev = fn(idx, table_wide, prev)
        wides.append(prev)

    acc = _merge_first(wides[0], seq_s, seq, batch, dim)
    for k in range(1, n_slices):
        acc = _merge_slice(wides[k], acc, k, seq_s, seq, batch, dim)
    return acc.transpose(2, 0, 1)


# speedup vs baseline: 1.1032x; 1.1032x over previous
import functools

import jax
import jax.numpy as jnp
from jax import lax
from jax.experimental import pallas as pl
from jax.experimental.pallas import tpu as pltpu
from jax.experimental.pallas import tpu_sc as plsc

_N_SLICES = 2


def _transpose_pad_body(dim, t_ref, out_ref):
    out_ref[:, :dim] = t_ref[...].T


def _transpose_pad(table_t, block_rows=4096):
    dim, vocab = table_t.shape
    grid = pl.cdiv(vocab, block_rows)
    return pl.pallas_call(
        functools.partial(_transpose_pad_body, dim),
        grid=(grid,),
        in_specs=[pl.BlockSpec((dim, block_rows), lambda i: (0, i))],
        out_specs=pl.BlockSpec((block_rows, 384), lambda i: (i, 0)),
        out_shape=jax.ShapeDtypeStruct((vocab, 384), jnp.float32),
    )(table_t)


def _make_gather(slice_idx, n_workers, n_chunks, chunk, num_cores):
    mesh = plsc.VectorSubcoreMesh(core_axis_name="c", subcore_axis_name="s")

    @functools.partial(
        pl.kernel,
        mesh=mesh,
        out_type=jax.ShapeDtypeStruct((n_workers * n_chunks * chunk, 384),
                                      jnp.float32),
        scratch_types=[
            pltpu.VMEM((n_chunks, chunk), jnp.int32),
            pltpu.VMEM((2, chunk, 384), jnp.float32),
            pltpu.SemaphoreType.DMA,
            pltpu.SemaphoreType.DMA,
            pltpu.SemaphoreType.DMA,
            pltpu.SemaphoreType.DMA,
        ],
    )
    def gather_kernel(idx_hbm, table_hbm, prev_hbm, out_hbm, idx_v, row_v,
                      gsem0, gsem1, wsem0, wsem1):
        del prev_hbm
        wid = lax.axis_index("s") * num_cores + lax.axis_index("c")
        pltpu.sync_copy(idx_hbm.at[slice_idx * n_workers + wid], idx_v)
        base = wid * (n_chunks * chunk)
        gsems = (gsem0, gsem1)
        wsems = (wsem0, wsem1)
        gathers = [None, None]
        writes = [None, None]
        gathers[0] = pltpu.async_copy(
            table_hbm.at[idx_v.at[0]], row_v.at[0], gsems[0])
        for j in range(n_chunks):
            cur = j % 2
            nxt = 1 - cur
            if j + 1 < n_chunks:
                if writes[nxt] is not None:
                    writes[nxt].wait()
                gathers[nxt] = pltpu.async_copy(
                    table_hbm.at[idx_v.at[j + 1]], row_v.at[nxt], gsems[nxt])
            gathers[cur].wait()
            writes[cur] = pltpu.async_copy(
                row_v.at[cur], out_hbm.at[pl.ds(base + j * chunk, chunk)],
                wsems[cur])
        for w in writes:
            if w is not None:
                w.wait()

    return gather_kernel


def _transpose_merge_body(dim, wide_ref, _acc_ref, out_ref):
    out_ref[0, ...] = wide_ref[:, :dim].T


def _merge_slice(wide_k, acc, k, seq_s, seq, batch, dim, block_cols=4096):
    nb = batch // block_cols
    grid = (seq_s, nb)
    return pl.pallas_call(
        functools.partial(_transpose_merge_body, dim),
        grid=grid,
        in_specs=[
            pl.BlockSpec((block_cols, 384), lambda s, b: (s * nb + b, 0)),
            pl.BlockSpec(memory_space=pl.ANY),
        ],
        out_specs=pl.BlockSpec((1, dim, block_cols),
                               lambda s, b, k=k: (k * seq_s + s, 0, b)),
        out_shape=jax.ShapeDtypeStruct((seq, dim, batch), jnp.float32),
        input_output_aliases={1: 0},
    )(wide_k, acc)


def _merge_first_body(dim, k, seq_s, nb, wide_ref, out_ref):
    out_ref[0, ...] = wide_ref[:, :dim].T


def _merge_first(wide_k, seq_s, seq, batch, dim, block_cols=4096):
    nb = batch // block_cols
    grid = (seq_s, nb)
    return pl.pallas_call(
        functools.partial(_merge_first_body, dim, 0, seq_s, nb),
        grid=grid,
        in_specs=[pl.BlockSpec((block_cols, 384), lambda s, b: (s * nb + b, 0))],
        out_specs=pl.BlockSpec((1, dim, block_cols), lambda s, b: (s, 0, b)),
        out_shape=jax.ShapeDtypeStruct((seq, dim, batch), jnp.float32),
    )(wide_k)


def kernel(inputs, table):
    batch, seq = inputs.shape
    vocab, dim = table.shape
    total = batch * seq

    info = plsc.get_sparse_core_info()
    n_workers = info.num_cores * info.num_subcores
    n_slices = _N_SLICES
    per_worker = total // (n_slices * n_workers)
    chunk = 128
    n_chunks = per_worker // chunk
    seq_s = seq // n_slices

    idx = inputs.astype(jnp.int32).T.reshape(
        n_slices * n_workers, n_chunks, chunk)
    table_wide = _transpose_pad(table.T)

    wides = []
    prev = idx
    for k in range(n_slices):
        fn = _make_gather(k, n_workers, n_chunks, chunk, info.num_cores)
        prev = fn(idx, table_wide, prev)
        wides.append(prev)

    acc = _merge_first(wides[0], seq_s, seq, batch, dim)
    for k in range(1, n_slices):
        acc = _merge_slice(wides[k], acc, k, seq_s, seq, batch, dim)
    return acc.transpose(2, 0, 1)


# speedup vs baseline: 1.1097x; 1.0060x over previous
import functools

import jax
import jax.numpy as jnp
from jax import lax
from jax.experimental import pallas as pl
from jax.experimental.pallas import tpu as pltpu
from jax.experimental.pallas import tpu_sc as plsc

_N_SLICES = 2


def _transpose_pad_body(dim, t_ref, out_ref):
    out_ref[:, :dim] = t_ref[...].T


def _transpose_pad(table_t, block_rows=8192):
    dim, vocab = table_t.shape
    grid = pl.cdiv(vocab, block_rows)
    return pl.pallas_call(
        functools.partial(_transpose_pad_body, dim),
        grid=(grid,),
        in_specs=[pl.BlockSpec((dim, block_rows), lambda i: (0, i))],
        out_specs=pl.BlockSpec((block_rows, 384), lambda i: (i, 0)),
        out_shape=jax.ShapeDtypeStruct((vocab, 384), jnp.float32),
    )(table_t)


def _make_gather(slice_idx, n_workers, n_chunks, chunk, num_cores):
    mesh = plsc.VectorSubcoreMesh(core_axis_name="c", subcore_axis_name="s")

    @functools.partial(
        pl.kernel,
        mesh=mesh,
        out_type=jax.ShapeDtypeStruct((n_workers * n_chunks * chunk, 384),
                                      jnp.float32),
        scratch_types=[
            pltpu.VMEM((n_chunks, chunk), jnp.int32),
            pltpu.VMEM((2, chunk, 384), jnp.float32),
            pltpu.SemaphoreType.DMA,
            pltpu.SemaphoreType.DMA,
            pltpu.SemaphoreType.DMA,
            pltpu.SemaphoreType.DMA,
        ],
    )
    def gather_kernel(idx_hbm, table_hbm, prev_hbm, out_hbm, idx_v, row_v,
                      gsem0, gsem1, wsem0, wsem1):
        del prev_hbm
        wid = lax.axis_index("s") * num_cores + lax.axis_index("c")
        pltpu.sync_copy(idx_hbm.at[slice_idx * n_workers + wid], idx_v)
        base = wid * (n_chunks * chunk)
        gsems = (gsem0, gsem1)
        wsems = (wsem0, wsem1)
        gathers = [None, None]
        writes = [None, None]
        gathers[0] = pltpu.async_copy(
            table_hbm.at[idx_v.at[0]], row_v.at[0], gsems[0])
        for j in range(n_chunks):
            cur = j % 2
            nxt = 1 - cur
            if j + 1 < n_chunks:
                if writes[nxt] is not None:
                    writes[nxt].wait()
                gathers[nxt] = pltpu.async_copy(
                    table_hbm.at[idx_v.at[j + 1]], row_v.at[nxt], gsems[nxt])
            gathers[cur].wait()
            writes[cur] = pltpu.async_copy(
                row_v.at[cur], out_hbm.at[pl.ds(base + j * chunk, chunk)],
                wsems[cur])
        for w in writes:
            if w is not None:
                w.wait()

    return gather_kernel


def _transpose_merge_body(dim, wide_ref, _acc_ref, out_ref):
    out_ref[0, ...] = wide_ref[:, :dim].T


def _merge_slice(wide_k, acc, k, seq_s, seq, batch, dim, block_cols=4096):
    nb = batch // block_cols
    grid = (seq_s, nb)
    return pl.pallas_call(
        functools.partial(_transpose_merge_body, dim),
        grid=grid,
        in_specs=[
            pl.BlockSpec((block_cols, 384), lambda s, b: (s * nb + b, 0)),
            pl.BlockSpec(memory_space=pl.ANY),
        ],
        out_specs=pl.BlockSpec((1, dim, block_cols),
                               lambda s, b, k=k: (k * seq_s + s, 0, b)),
        out_shape=jax.ShapeDtypeStruct((seq, dim, batch), jnp.float32),
        input_output_aliases={1: 0},
    )(wide_k, acc)


def _merge_first_body(dim, k, seq_s, nb, wide_ref, out_ref):
    out_ref[0, ...] = wide_ref[:, :dim].T


def _merge_first(wide_k, seq_s, seq, batch, dim, block_cols=4096):
    nb = batch // block_cols
    grid = (seq_s, nb)
    return pl.pallas_call(
        functools.partial(_merge_first_body, dim, 0, seq_s, nb),
        grid=grid,
        in_specs=[pl.BlockSpec((block_cols, 384), lambda s, b: (s * nb + b, 0))],
        out_specs=pl.BlockSpec((1, dim, block_cols), lambda s, b: (s, 0, b)),
        out_shape=jax.ShapeDtypeStruct((seq, dim, batch), jnp.float32),
    )(wide_k)


def kernel(inputs, table):
    batch, seq = inputs.shape
    vocab, dim = table.shape
    total = batch * seq

    info = plsc.get_sparse_core_info()
    n_workers = info.num_cores * info.num_subcores
    n_slices = _N_SLICES
    per_worker = total // (n_slices * n_workers)
    chunk = 128
    n_chunks = per_worker // chunk
    seq_s = seq // n_slices

    idx = inputs.astype(jnp.int32).T.reshape(
        n_slices * n_workers, n_chunks, chunk)
    table_wide = _transpose_pad(table.T)

    wides = []
    prev = idx
    for k in range(n_slices):
        fn = _make_gather(k, n_workers, n_chunks, chunk, info.num_cores)
        prev = fn(idx, table_wide, prev)
        wides.append(prev)

    acc = _merge_first(wides[0], seq_s, seq, batch, dim)
    for k in range(1, n_slices):
        acc = _merge_slice(wides[k], acc, k, seq_s, seq, batch, dim)
    return acc.transpose(2, 0, 1)


# speedup vs baseline: 1.1155x; 1.0052x over previous
import functools

import jax
import jax.numpy as jnp
from jax import lax
from jax.experimental import pallas as pl
from jax.experimental.pallas import tpu as pltpu
from jax.experimental.pallas import tpu_sc as plsc

_N_SLICES = 2


def _transpose_pad_body(dim, t_ref, out_ref):
    out_ref[:, :dim] = t_ref[...].T


def _transpose_pad(table_t, block_rows=8192):
    dim, vocab = table_t.shape
    grid = pl.cdiv(vocab, block_rows)
    return pl.pallas_call(
        functools.partial(_transpose_pad_body, dim),
        grid=(grid,),
        in_specs=[pl.BlockSpec((dim, block_rows), lambda i: (0, i))],
        out_specs=pl.BlockSpec((block_rows, 384), lambda i: (i, 0)),
        out_shape=jax.ShapeDtypeStruct((vocab, 384), jnp.float32),
    )(table_t)


def _make_gather(slice_idx, n_workers, n_chunks, chunk, num_cores):
    mesh = plsc.VectorSubcoreMesh(core_axis_name="c", subcore_axis_name="s")

    @functools.partial(
        pl.kernel,
        mesh=mesh,
        out_type=jax.ShapeDtypeStruct((n_workers * n_chunks * chunk, 384),
                                      jnp.float32),
        scratch_types=[
            pltpu.VMEM((n_chunks, chunk), jnp.int32),
            pltpu.VMEM((2, chunk, 384), jnp.float32),
            pltpu.SemaphoreType.DMA,
            pltpu.SemaphoreType.DMA,
            pltpu.SemaphoreType.DMA,
            pltpu.SemaphoreType.DMA,
        ],
    )
    def gather_kernel(idx_hbm, table_hbm, prev_hbm, out_hbm, idx_v, row_v,
                      gsem0, gsem1, wsem0, wsem1):
        del prev_hbm
        wid = lax.axis_index("s") * num_cores + lax.axis_index("c")
        pltpu.sync_copy(idx_hbm.at[slice_idx * n_workers + wid], idx_v)
        base = wid * (n_chunks * chunk)
        gsems = (gsem0, gsem1)
        wsems = (wsem0, wsem1)
        gathers = [None, None]
        writes = [None, None]
        gathers[0] = pltpu.async_copy(
            table_hbm.at[idx_v.at[0]], row_v.at[0], gsems[0])
        for j in range(n_chunks):
            cur = j % 2
            nxt = 1 - cur
            if j + 1 < n_chunks:
                if writes[nxt] is not None:
                    writes[nxt].wait()
                gathers[nxt] = pltpu.async_copy(
                    table_hbm.at[idx_v.at[j + 1]], row_v.at[nxt], gsems[nxt])
            gathers[cur].wait()
            writes[cur] = pltpu.async_copy(
                row_v.at[cur], out_hbm.at[pl.ds(base + j * chunk, chunk)],
                wsems[cur])
        for w in writes:
            if w is not None:
                w.wait()

    return gather_kernel


def _transpose_merge_body(dim, batch, wide_ref, _acc_ref, out_ref):
    out_ref[0, ...] = wide_ref[:batch, :dim].T
    out_ref[1, ...] = wide_ref[batch:, :dim].T


def _merge_slice(wide_k, acc, k, seq_s, seq, batch, dim):
    grid = (seq_s // 2,)
    return pl.pallas_call(
        functools.partial(_transpose_merge_body, dim, batch),
        grid=grid,
        in_specs=[
            pl.BlockSpec((2 * batch, 384), lambda s: (s, 0)),
            pl.BlockSpec(memory_space=pl.ANY),
        ],
        out_specs=pl.BlockSpec((2, dim, batch),
                               lambda s, k=k: (k * seq_s // 2 + s, 0, 0)),
        out_shape=jax.ShapeDtypeStruct((seq, dim, batch), jnp.float32),
        input_output_aliases={1: 0},
    )(wide_k, acc)


def _merge_first_body(dim, batch, wide_ref, out_ref):
    out_ref[0, ...] = wide_ref[:batch, :dim].T
    out_ref[1, ...] = wide_ref[batch:, :dim].T


def _merge_first(wide_k, seq_s, seq, batch, dim):
    grid = (seq_s // 2,)
    return pl.pallas_call(
        functools.partial(_merge_first_body, dim, batch),
        grid=grid,
        in_specs=[pl.BlockSpec((2 * batch, 384), lambda s: (s, 0))],
        out_specs=pl.BlockSpec((2, dim, batch), lambda s: (s, 0, 0)),
        out_shape=jax.ShapeDtypeStruct((seq, dim, batch), jnp.float32),
    )(wide_k)


def kernel(inputs, table):
    batch, seq = inputs.shape
    vocab, dim = table.shape
    total = batch * seq

    info = plsc.get_sparse_core_info()
    n_workers = info.num_cores * info.num_subcores
    n_slices = _N_SLICES
    per_worker = total // (n_slices * n_workers)
    chunk = 128
    n_chunks = per_worker // chunk
    seq_s = seq // n_slices

    idx = inputs.astype(jnp.int32).T.reshape(
        n_slices * n_workers, n_chunks, chunk)
    table_wide = _transpose_pad(table.T)

    wides = []
    prev = idx
    for k in range(n_slices):
        fn = _make_gather(k, n_workers, n_chunks, chunk, info.num_cores)
        prev = fn(idx, table_wide, prev)
        wides.append(prev)

    acc = _merge_first(wides[0], seq_s, seq, batch, dim)
    for k in range(1, n_slices):
        acc = _merge_slice(wides[k], acc, k, seq_s, seq, batch, dim)
    return acc.transpose(2, 0, 1)


# speedup vs baseline: 1.1206x; 1.0046x over previous
import functools

import jax
import jax.numpy as jnp
from jax import lax
from jax.experimental import pallas as pl
from jax.experimental.pallas import tpu as pltpu
from jax.experimental.pallas import tpu_sc as plsc

_N_SLICES = 1


def _transpose_pad_body(dim, t_ref, out_ref):
    out_ref[:, :dim] = t_ref[...].T


def _transpose_pad(table_t, block_rows=8192):
    dim, vocab = table_t.shape
    grid = pl.cdiv(vocab, block_rows)
    return pl.pallas_call(
        functools.partial(_transpose_pad_body, dim),
        grid=(grid,),
        in_specs=[pl.BlockSpec((dim, block_rows), lambda i: (0, i))],
        out_specs=pl.BlockSpec((block_rows, 384), lambda i: (i, 0)),
        out_shape=jax.ShapeDtypeStruct((vocab, 384), jnp.float32),
    )(table_t)


def _make_gather(slice_idx, n_workers, n_chunks, chunk, num_cores):
    mesh = plsc.VectorSubcoreMesh(core_axis_name="c", subcore_axis_name="s")

    @functools.partial(
        pl.kernel,
        mesh=mesh,
        out_type=jax.ShapeDtypeStruct((n_workers * n_chunks * chunk, 384),
                                      jnp.float32),
        scratch_types=[
            pltpu.VMEM((n_chunks, chunk), jnp.int32),
            pltpu.VMEM((2, chunk, 384), jnp.float32),
            pltpu.SemaphoreType.DMA,
            pltpu.SemaphoreType.DMA,
            pltpu.SemaphoreType.DMA,
            pltpu.SemaphoreType.DMA,
        ],
    )
    def gather_kernel(idx_hbm, table_hbm, prev_hbm, out_hbm, idx_v, row_v,
                      gsem0, gsem1, wsem0, wsem1):
        del prev_hbm
        wid = lax.axis_index("s") * num_cores + lax.axis_index("c")
        pltpu.sync_copy(idx_hbm.at[slice_idx * n_workers + wid], idx_v)
        base = wid * (n_chunks * chunk)
        gsems = (gsem0, gsem1)
        wsems = (wsem0, wsem1)
        gathers = [None, None]
        writes = [None, None]
        gathers[0] = pltpu.async_copy(
            table_hbm.at[idx_v.at[0]], row_v.at[0], gsems[0])
        for j in range(n_chunks):
            cur = j % 2
            nxt = 1 - cur
            if j + 1 < n_chunks:
                if writes[nxt] is not None:
                    writes[nxt].wait()
                gathers[nxt] = pltpu.async_copy(
                    table_hbm.at[idx_v.at[j + 1]], row_v.at[nxt], gsems[nxt])
            gathers[cur].wait()
            writes[cur] = pltpu.async_copy(
                row_v.at[cur], out_hbm.at[pl.ds(base + j * chunk, chunk)],
                wsems[cur])
        for w in writes:
            if w is not None:
                w.wait()

    return gather_kernel


def _transpose_merge_body(dim, batch, wide_ref, _acc_ref, out_ref):
    out_ref[0, ...] = wide_ref[:batch, :dim].T
    out_ref[1, ...] = wide_ref[batch:, :dim].T


def _merge_slice(wide_k, acc, k, seq_s, seq, batch, dim):
    grid = (seq_s // 2,)
    return pl.pallas_call(
        functools.partial(_transpose_merge_body, dim, batch),
        grid=grid,
        in_specs=[
            pl.BlockSpec((2 * batch, 384), lambda s: (s, 0)),
            pl.BlockSpec(memory_space=pl.ANY),
        ],
        out_specs=pl.BlockSpec((2, dim, batch),
                               lambda s, k=k: (k * seq_s // 2 + s, 0, 0)),
        out_shape=jax.ShapeDtypeStruct((seq, dim, batch), jnp.float32),
        input_output_aliases={1: 0},
    )(wide_k, acc)


def _merge_first_body(dim, batch, wide_ref, out_ref):
    out_ref[0, ...] = wide_ref[:batch, :dim].T
    out_ref[1, ...] = wide_ref[batch:, :dim].T


def _merge_first(wide_k, seq_s, seq, batch, dim):
    grid = (seq_s // 2,)
    return pl.pallas_call(
        functools.partial(_merge_first_body, dim, batch),
        grid=grid,
        in_specs=[pl.BlockSpec((2 * batch, 384), lambda s: (s, 0))],
        out_specs=pl.BlockSpec((2, dim, batch), lambda s: (s, 0, 0)),
        out_shape=jax.ShapeDtypeStruct((seq, dim, batch), jnp.float32),
    )(wide_k)


def kernel(inputs, table):
    batch, seq = inputs.shape
    vocab, dim = table.shape
    total = batch * seq

    info = plsc.get_sparse_core_info()
    n_workers = info.num_cores * info.num_subcores
    n_slices = _N_SLICES
    per_worker = total // (n_slices * n_workers)
    chunk = 128
    n_chunks = per_worker // chunk
    seq_s = seq // n_slices

    idx = inputs.astype(jnp.int32).T.reshape(
        n_slices * n_workers, n_chunks, chunk)
    table_wide = _transpose_pad(table.T)

    wides = []
    prev = idx
    for k in range(n_slices):
        fn = _make_gather(k, n_workers, n_chunks, chunk, info.num_cores)
        prev = fn(idx, table_wide, prev)
        wides.append(prev)

    acc = _merge_first(wides[0], seq_s, seq, batch, dim)
    for k in range(1, n_slices):
        acc = _merge_slice(wides[k], acc, k, seq_s, seq, batch, dim)
    return acc.transpose(2, 0, 1)
